# baseline (speedup 1.0000x reference)
import jax
import jax.numpy as jnp
from jax.experimental import pallas as pl
from jax.experimental.pallas import tpu as pltpu

N = 8192
D = 128
BM0 = 512
BM = 1024


def _layer0_kernel(a_ref, x_ref, w0_ref, w1_ref, o_ref, a16_ref, y1_ref, y0_scr):
    @pl.when(pl.program_id(0) == 0)
    def _prologue():
        y0 = jnp.dot(x_ref[...], w0_ref[...], preferred_element_type=jnp.float32)
        y0_scr[...] = y0.astype(jnp.bfloat16)

    a16 = a_ref[...].astype(jnp.bfloat16)
    a16_ref[...] = a16
    h = jnp.maximum(
        jnp.dot(a16, y0_scr[...], preferred_element_type=jnp.float32), 0.0)
    o_ref[...] = h
    y1 = jnp.dot(h, w1_ref[...], preferred_element_type=jnp.float32)
    y1_ref[...] = y1.astype(jnp.bfloat16)


def _layer0(a, x, w0, w1):
    return pl.pallas_call(
        _layer0_kernel,
        grid=(N // BM0,),
        in_specs=[
            pl.BlockSpec((BM0, N), lambda i: (i, 0)),
            pl.BlockSpec((N, D), lambda i: (0, 0)),
            pl.BlockSpec((D, D), lambda i: (0, 0)),
            pl.BlockSpec((D, D), lambda i: (0, 0)),
        ],
        out_specs=[
            pl.BlockSpec((BM0, D), lambda i: (i, 0)),
            pl.BlockSpec((BM0, N), lambda i: (i, 0)),
            pl.BlockSpec((BM0, D), lambda i: (i, 0)),
        ],
        out_shape=[
            jax.ShapeDtypeStruct((N, 3 * D), jnp.float32),
            jax.ShapeDtypeStruct((N, N), jnp.bfloat16),
            jax.ShapeDtypeStruct((N, D), jnp.bfloat16),
        ],
        scratch_shapes=[pltpu.VMEM((N, D), jnp.bfloat16)],
    )(a, x, w0, w1)


NSPLIT = 4
KC = N // NSPLIT


def _acc_chunks(a_refs, y_ref):
    acc = None
    for k, a_ref in enumerate(a_refs):
        p = jnp.dot(a_ref[...], y_ref[k * KC:(k + 1) * KC, :],
                    preferred_element_type=jnp.float32)
        acc = p if acc is None else acc + p
    return acc


def _layer1_kernel(a0, a1, a2, a3, y_ref, w_ref, o_in_ref, o_ref, ynext_ref):
    del o_in_ref
    h = jnp.maximum(_acc_chunks((a0, a1, a2, a3), y_ref), 0.0)
    o_ref[...] = h
    ynext = jnp.dot(h, w_ref[...], preferred_element_type=jnp.float32)
    ynext_ref[...] = ynext.astype(jnp.bfloat16)


def _chunk_specs():
    return [pl.BlockSpec((BM, KC), (lambda i, k=k: (i, k)))
            for k in range(NSPLIT)]


def _layer1(a16, y, w, o):
    return pl.pallas_call(
        _layer1_kernel,
        grid=(N // BM,),
        in_specs=_chunk_specs() + [
            pl.BlockSpec((N, D), lambda i: (0, 0)),
            pl.BlockSpec((D, D), lambda i: (0, 0)),
            pl.BlockSpec(memory_space=pl.ANY),
        ],
        out_specs=[
            pl.BlockSpec((BM, D), lambda i: (i, 1)),
            pl.BlockSpec((BM, D), lambda i: (i, 0)),
        ],
        out_shape=[
            jax.ShapeDtypeStruct((N, 3 * D), jnp.float32),
            jax.ShapeDtypeStruct((N, D), jnp.bfloat16),
        ],
        input_output_aliases={6: 0},
    )(a16, a16, a16, a16, y, w, o)


def _layer2_kernel(a0, a1, a2, a3, y_ref, o_in_ref, o_ref):
    del o_in_ref
    o_ref[...] = jnp.maximum(_acc_chunks((a0, a1, a2, a3), y_ref), 0.0)


def _layer2(a16, y, o):
    return pl.pallas_call(
        _layer2_kernel,
        grid=(N // BM,),
        in_specs=_chunk_specs() + [
            pl.BlockSpec((N, D), lambda i: (0, 0)),
            pl.BlockSpec(memory_space=pl.ANY),
        ],
        out_specs=pl.BlockSpec((BM, D), lambda i: (i, 2)),
        out_shape=jax.ShapeDtypeStruct((N, 3 * D), jnp.float32),
        input_output_aliases={5: 0},
    )(a16, a16, a16, a16, y, o)


def kernel(x, adj_norm, W0, W1, W2):
    o1, a16, y1 = _layer0(adj_norm, x, W0, W1)
    o2, y2 = _layer1(a16, y1, W2, o1)
    return _layer2(a16, y2, o2)


# speedup vs baseline: 1.0349x; 1.0349x over previous
import jax
import jax.numpy as jnp
from jax.experimental import pallas as pl
from jax.experimental.pallas import tpu as pltpu

N = 8192
D = 128
BM0 = 512
BM = 1024


def _layer0_kernel(a_ref, x_ref, w0_ref, w1_ref, o_ref, a16_ref, y1_ref, y0_scr):
    @pl.when(pl.program_id(0) == 0)
    def _prologue():
        y0 = jnp.dot(x_ref[...], w0_ref[...], preferred_element_type=jnp.float32)
        y0_scr[...] = y0.astype(jnp.bfloat16)

    a16 = a_ref[...].astype(jnp.bfloat16)
    a16_ref[...] = a16
    h = jnp.maximum(
        jnp.dot(a16, y0_scr[...], preferred_element_type=jnp.float32), 0.0)
    o_ref[...] = h
    y1 = jnp.dot(h, w1_ref[...], preferred_element_type=jnp.float32)
    y1_ref[...] = y1.astype(jnp.bfloat16)


def _layer0(a, x, w0, w1):
    return pl.pallas_call(
        _layer0_kernel,
        grid=(N // BM0,),
        in_specs=[
            pl.BlockSpec((BM0, N), lambda i: (i, 0)),
            pl.BlockSpec((N, D), lambda i: (0, 0)),
            pl.BlockSpec((D, D), lambda i: (0, 0)),
            pl.BlockSpec((D, D), lambda i: (0, 0)),
        ],
        out_specs=[
            pl.BlockSpec((BM0, D), lambda i: (i, 0)),
            pl.BlockSpec((BM0, N), lambda i: (i, 0)),
            pl.BlockSpec((BM0, D), lambda i: (i, 0)),
        ],
        out_shape=[
            jax.ShapeDtypeStruct((N, 3 * D), jnp.float32),
            jax.ShapeDtypeStruct((N, N), jnp.bfloat16),
            jax.ShapeDtypeStruct((N, D), jnp.bfloat16),
        ],
        scratch_shapes=[pltpu.VMEM((N, D), jnp.bfloat16)],
    )(a, x, w0, w1)


def _layers12_kernel(a16_ref, y1_ref, w2_ref, o_in_ref, o_ref, y_scr):
    del o_in_ref
    l = pl.program_id(0)
    i = pl.program_id(1)

    @pl.when((l == 0) & (i == 0))
    def _prologue():
        y_scr[0] = y1_ref[...]

    h = jnp.maximum(
        jnp.dot(a16_ref[...], y_scr[l], preferred_element_type=jnp.float32),
        0.0)
    o_ref[...] = h

    @pl.when(l == 0)
    def _epilogue():
        y2 = jnp.dot(h, w2_ref[...], preferred_element_type=jnp.float32)
        y_scr[1, pl.ds(i * BM, BM), :] = y2.astype(jnp.bfloat16)


def _layers12(a16, y1, w2, o):
    return pl.pallas_call(
        _layers12_kernel,
        grid=(2, N // BM),
        in_specs=[
            pl.BlockSpec((BM, N), lambda l, i: (i, 0)),
            pl.BlockSpec((N, D), lambda l, i: (0, 0)),
            pl.BlockSpec((D, D), lambda l, i: (0, 0)),
            pl.BlockSpec(memory_space=pl.ANY),
        ],
        out_specs=pl.BlockSpec((BM, D), lambda l, i: (i, 1 + l)),
        out_shape=jax.ShapeDtypeStruct((N, 3 * D), jnp.float32),
        input_output_aliases={3: 0},
        scratch_shapes=[pltpu.VMEM((2, N, D), jnp.bfloat16)],
    )(a16, y1, w2, o)


def kernel(x, adj_norm, W0, W1, W2):
    o1, a16, y1 = _layer0(adj_norm, x, W0, W1)
    return _layers12(a16, y1, W2, o1)
